# baseline (speedup 1.0000x reference)
import functools

import jax
import jax.numpy as jnp
from jax import lax
from jax.experimental import pallas as pl
from jax.experimental.pallas import tpu as pltpu
from jax.experimental.pallas import tpu_sc as plsc

B = 16384
D = 64
NC = 2
NS = 16
NW = NC * NS
BPW = B // NW
CHUNK = 128
NCHUNK = BPW // CHUNK
L = 16
VROWS = 500000
BROWS = 7816


def _mf_body(uidx_hbm, iidx_hbm, uemb_hbm, iemb_hbm, ubias_hbm, ibias_hbm,
             out_hbm,
             uidx_v, iidx_v, gidx_u, gidx_i, bidx_u, bidx_i,
             uslab, islab, ubslab, ibslab, out_v, sem):
    wid = lax.axis_index("s") * NC + lax.axis_index("c")

    lane = lax.broadcasted_iota(jnp.int32, (L,), 0)
    onehots = [(lane == j).astype(jnp.float32) for j in range(L)]

    for k in range(NCHUNK):
        q = wid * NCHUNK + k
        a = q // 8
        b = q % 8
        pltpu.sync_copy(uidx_hbm.at[a, b], uidx_v)
        pltpu.sync_copy(iidx_hbm.at[a, b], iidx_v)

        for t in range(CHUNK // L):
            sl = pl.ds(t * L, L)
            uv = uidx_v[sl]
            iv = iidx_v[sl]
            gidx_u[sl] = lax.shift_right_logical(uv, 1)
            gidx_i[sl] = lax.shift_right_logical(iv, 1)
            bidx_u[sl] = lax.shift_right_logical(uv, 7)
            bidx_i[sl] = lax.shift_right_logical(iv, 7)

        cps = [
            pltpu.async_copy(uemb_hbm.at[gidx_u], uslab, sem),
            pltpu.async_copy(iemb_hbm.at[gidx_i], islab, sem),
            pltpu.async_copy(ubias_hbm.at[bidx_u], ubslab, sem),
            pltpu.async_copy(ibias_hbm.at[bidx_i], ibslab, sem),
        ]
        for c in cps:
            c.wait()

        def group(g, carry):
            sl = pl.ds(g * L, L)
            evec = g * L + lane
            uv = uidx_v[sl]
            iv = iidx_v[sl]
            accv = (plsc.load_gather(ubslab, [evec, uv & 127])
                    + plsc.load_gather(ibslab, [evec, iv & 127]))
            uoff = (uv & 1) * D
            ioff = (iv & 1) * D
            for j in range(L):
                e = g * L + j
                ou = uoff[j]
                oi = ioff[j]
                prod = (uslab[e, pl.ds(ou, L)] * islab[e, pl.ds(oi, L)])
                for c in range(1, D // L):
                    prod = prod + (uslab[e, pl.ds(ou + c * L, L)]
                                   * islab[e, pl.ds(oi + c * L, L)])
                s = jnp.sum(prod)
                accv = accv + jnp.broadcast_to(s, (L,)) * onehots[j]
            out_v[pl.ds(k * CHUNK + g * L, L)] = accv
            return carry

        lax.fori_loop(0, CHUNK // L, group, 0)

    pltpu.sync_copy(out_v, out_hbm.at[pl.ds(wid * BPW, BPW)])


@jax.jit
def kernel(u_idx, i_idx, u_emb, i_emb, u_bias, i_bias):
    mesh = plsc.VectorSubcoreMesh(core_axis_name="c", subcore_axis_name="s")
    run = functools.partial(
        pl.kernel,
        mesh=mesh,
        compiler_params=pltpu.CompilerParams(needs_layout_passes=False),
        out_type=jax.ShapeDtypeStruct((B,), jnp.float32),
        scratch_types=[
            pltpu.VMEM((CHUNK,), jnp.int32),
            pltpu.VMEM((CHUNK,), jnp.int32),
            pltpu.VMEM((CHUNK,), jnp.int32),
            pltpu.VMEM((CHUNK,), jnp.int32),
            pltpu.VMEM((CHUNK,), jnp.int32),
            pltpu.VMEM((CHUNK,), jnp.int32),
            pltpu.VMEM((CHUNK, 2 * D), jnp.float32),
            pltpu.VMEM((CHUNK, 2 * D), jnp.float32),
            pltpu.VMEM((CHUNK, 2 * D), jnp.float32),
            pltpu.VMEM((CHUNK, 2 * D), jnp.float32),
            pltpu.VMEM((BPW,), jnp.float32),
            pltpu.SemaphoreType.DMA,
        ],
    )(_mf_body)
    uidx3 = u_idx.reshape(B // (8 * CHUNK), 8, CHUNK)
    iidx3 = i_idx.reshape(B // (8 * CHUNK), 8, CHUNK)
    uemb2 = u_emb.reshape(VROWS, 2 * D)
    iemb2 = i_emb.reshape(VROWS, 2 * D)
    pad = BROWS * 2 * D - u_bias.size
    ub2 = jnp.pad(u_bias.reshape(-1), (0, pad)).reshape(BROWS, 2 * D)
    ib2 = jnp.pad(i_bias.reshape(-1), (0, pad)).reshape(BROWS, 2 * D)
    return run(uidx3, iidx3, uemb2, iemb2, ub2, ib2)
